# baseline (speedup 1.0000x reference)
import jax
import jax.numpy as jnp
from jax import lax
from jax.experimental import pallas as pl
from jax.experimental.pallas import tpu as pltpu
from jax.experimental.pallas import tpu_sc as plsc

NN = 10000
EE = 320000
DD = 128
NC = 2
NS = 16
NW = NC * NS
CH = 128
NCHUNK = 80
NCK0 = 112
NCK1 = 48
NCKMAX = max(NCK0, NCK1)
EPT = CH * NCHUNK
EPAD = EPT * NW
NPAD = 10240
ROWS_PER_TILE = NPAD // NS
DUMMY = NN
DW = 128

def _deg_body(dst_hbm, out_hbm, dst_v, obuf, zbuf, acc):
    c = lax.axis_index("c")
    s = lax.axis_index("s")
    wid = c * NS + s

    def obody(r, carry):
        for q in range(DW // 16):
            obuf[r, pl.ds(q * 16, 16)] = jnp.ones((16,), jnp.float32)
        return carry

    lax.fori_loop(0, CH, obody, 0)
    for r in range(16):
        for q in range(DW // 16):
            zbuf[r, pl.ds(q * 16, 16)] = jnp.zeros((16,), jnp.float32)
    base = s * ROWS_PER_TILE

    def zbody(k, carry):
        pltpu.sync_copy(zbuf, acc.at[pl.ds(base + k * 16, 16)])
        return carry

    lax.fori_loop(0, ROWS_PER_TILE // 16, zbody, 0)
    pltpu.sync_copy(dst_hbm.at[wid], dst_v)
    plsc.subcore_barrier()

    def body(j, carry):
        pltpu.sync_copy(obuf, acc.at[dst_v.at[j]], add=True)
        return carry

    lax.fori_loop(0, NCHUNK, body, 0)
    plsc.subcore_barrier()
    pltpu.sync_copy(acc.at[pl.ds(base, ROWS_PER_TILE)],
                    out_hbm.at[c, pl.ds(base, ROWS_PER_TILE)])


import functools


@functools.lru_cache(maxsize=None)
def _get_mesh():
    return plsc.VectorSubcoreMesh(core_axis_name="c", subcore_axis_name="s",
                                  num_cores=NC, num_subcores=NS)


@functools.lru_cache(maxsize=None)
def _get_deg_kernel():
    return pl.kernel(
        _deg_body,
        out_type=jax.ShapeDtypeStruct((NC, NPAD, DW), jnp.float32),
        mesh=_get_mesh(),
        scratch_types=[
            pltpu.VMEM((NCHUNK, CH), jnp.int32),
            pltpu.VMEM((CH, DW), jnp.float32),
            pltpu.VMEM((16, DW), jnp.float32),
            pltpu.VMEM_SHARED((NPAD, DW), jnp.float32),
        ],
    )


NBUF = 2
NIDX = 4


def _seg_body(u_hbm, src_hbm, dst_hbm, out_hbm, srcr, dstr, ssrc, sdst,
              gbuf0, gbuf1, zbuf, acc,
              gs0, gs1, ss0, ss1,
              isem0, isem1, isem2, isem3,
              jsem0, jsem1, jsem2, jsem3):
    gbufs = (gbuf0, gbuf1)
    gsems = (gs0, gs1)
    ssems = (ss0, ss1)
    isems = (isem0, isem1, isem2, isem3)
    jsems = (jsem0, jsem1, jsem2, jsem3)
    c = lax.axis_index("c")
    s = lax.axis_index("s")
    wid = c * NS + s
    nck = jnp.where(c == 0, NCK0, NCK1)
    for r in range(8):
        for q in range(DD // 16):
            zbuf[r, pl.ds(q * 16, 16)] = jnp.zeros((16,), jnp.float32)
    base = s * ROWS_PER_TILE

    def zbody(k, carry):
        pltpu.sync_copy(zbuf, acc.at[pl.ds(base + k * 8, 8)])
        return carry

    lax.fori_loop(0, ROWS_PER_TILE // 8, zbody, 0)

    @pl.when(c == 0)
    def _pipelined():
        for q in range(NIDX - 1):
            pltpu.async_copy(src_hbm.at[wid, q], srcr.at[q], isems[q])
            pltpu.async_copy(dst_hbm.at[wid, q], dstr.at[q], jsems[q])
        pltpu.make_async_copy(src_hbm.at[wid, 0], srcr.at[0],
                              isems[0]).wait()
        plsc.subcore_barrier()
        pltpu.async_copy(u_hbm.at[srcr.at[0]], gbufs[0], gsems[0])

        def step(t, carry):
            for u in range(2 * NBUF):
                j = t * 2 * NBUF + u
                b = u % NBUF
                pb = (u - 1) % NBUF
                pq = (u - 1) % NIDX
                q1 = (u + 1) % NIDX
                q3 = (u + 3) % NIDX
                pltpu.make_async_copy(u_hbm.at[srcr.at[u]], gbufs[b],
                                      gsems[b]).wait()

                @pl.when(j > 0)
                def _():
                    pltpu.make_async_copy(gbufs[pb],
                                          acc.at[dstr.at[pq]],
                                          ssems[pb]).wait()

                pltpu.make_async_copy(dst_hbm.at[wid, j], dstr.at[u],
                                      jsems[u]).wait()
                pltpu.async_copy(gbufs[b], acc.at[dstr.at[u]], ssems[b],
                                 add=True)

                @pl.when(j + 1 < nck)
                def _():
                    pltpu.make_async_copy(src_hbm.at[wid, j + 1],
                                          srcr.at[q1], isems[q1]).wait()
                    pltpu.async_copy(u_hbm.at[srcr.at[q1]], gbufs[pb],
                                     gsems[pb])

                @pl.when(j + 3 < nck)
                def _():
                    pltpu.async_copy(src_hbm.at[wid, j + 3], srcr.at[q3],
                                     isems[q3])
                    pltpu.async_copy(dst_hbm.at[wid, j + 3], dstr.at[q3],
                                     jsems[q3])
            return carry

        lax.fori_loop(0, nck // (2 * NBUF), step, 0)
        pltpu.make_async_copy(gbufs[1], acc.at[dstr.at[3]],
                              ssems[1]).wait()

    @pl.when(c != 0)
    def _serial():
        pltpu.sync_copy(src_hbm.at[wid, pl.ds(0, NCK1)], ssrc)
        pltpu.sync_copy(dst_hbm.at[wid, pl.ds(0, NCK1)], sdst)
        plsc.subcore_barrier()

        def sstep(j, carry):
            pltpu.async_copy(u_hbm.at[ssrc.at[j]], gbufs[0],
                             gsems[0]).wait()
            pltpu.sync_copy(gbufs[0], acc.at[sdst.at[j]], add=True)
            return carry

        lax.fori_loop(0, NCK1, sstep, 0)

    plsc.subcore_barrier()
    pltpu.sync_copy(acc.at[pl.ds(base, ROWS_PER_TILE)],
                    out_hbm.at[c, pl.ds(base, ROWS_PER_TILE)])


@functools.lru_cache(maxsize=None)
def _get_seg_kernel():
    return pl.kernel(
        _seg_body,
        out_type=jax.ShapeDtypeStruct((NC, NPAD, DD), jnp.float32),
        mesh=_get_mesh(),
        scratch_types=(
            [pltpu.VMEM((NIDX, CH), jnp.int32)] * 2
            + [pltpu.VMEM((NCK1, CH), jnp.int32)] * 2
            + [pltpu.VMEM((CH, DD), jnp.float32)] * NBUF
            + [pltpu.VMEM((8, DD), jnp.float32)]
            + [pltpu.VMEM_SHARED((NPAD, DD), jnp.float32)]
            + [pltpu.SemaphoreType.DMA] * (2 * NBUF + 2 * NIDX)
        ),
    )


def _prep_body(deg_ref, x_ref, w_ref, dis_ref, u_ref):
    deg = deg_ref[0, :, 0:1] + deg_ref[1, :, 0:1] + 1.0
    row = lax.broadcasted_iota(jnp.int32, (NPAD, 1), 0)
    dis = jnp.where(row < NN, lax.rsqrt(deg), 0.0)
    dis_ref[...] = dis
    u_ref[...] = dis * jnp.dot(x_ref[...], w_ref[...],
                               preferred_element_type=jnp.float32)


_prep = pl.pallas_call(
    _prep_body,
    out_shape=(jax.ShapeDtypeStruct((NPAD, 1), jnp.float32),
               jax.ShapeDtypeStruct((NPAD, DD), jnp.float32)),
)


def _mid_body(s_ref, u_ref, dis_ref, b_ref, g_ref, be_ref, a_ref, w_ref,
              out_ref):
    dis = dis_ref[...]
    y = dis * (s_ref[0] + s_ref[1] + u_ref[...]) + b_ref[...]
    row = lax.broadcasted_iota(jnp.int32, (NPAD, 1), 0)
    mask = row < NN
    ym = jnp.where(mask, y, 0.0)
    m = jnp.sum(ym, axis=0, keepdims=True) * (1.0 / NN)
    d = jnp.where(mask, y - m, 0.0)
    v = jnp.sum(d * d, axis=0, keepdims=True) * (1.0 / NN)
    z = g_ref[...] * d * lax.rsqrt(v + 1e-5) + be_ref[...]
    a = a_ref[0]
    z = jnp.maximum(z, 0.0) + a * jnp.minimum(z, 0.0)
    out_ref[...] = dis * jnp.dot(z, w_ref[...],
                                 preferred_element_type=jnp.float32)


_mid = pl.pallas_call(
    _mid_body,
    out_shape=jax.ShapeDtypeStruct((NPAD, DD), jnp.float32),
)


def _final_body(s_ref, u_ref, dis_ref, b_ref, out_ref):
    y = dis_ref[...] * (s_ref[0] + s_ref[1] + u_ref[...]) + b_ref[...]
    out_ref[...] = y[:NN, :]


_final = pl.pallas_call(
    _final_body,
    out_shape=jax.ShapeDtypeStruct((NN, DD), jnp.float32),
)


def kernel(x, edge_index, W1, b1, g1, be1, a1, W2, b2, g2, be2, a2, W3, b3):
    src = edge_index[0]
    dst = edge_index[1]
    pad = EPAD - EE
    srcf = jnp.concatenate([src, jnp.zeros((pad,), jnp.int32)])
    dstf = jnp.concatenate(
        [dst, DUMMY + (jnp.arange(pad, dtype=jnp.int32) % (NPAD - NN))])
    dstp_deg = dstf.reshape(NW, NCHUNK, CH)
    e0 = NS * NCK0 * CH
    cpad0 = ((0, 0), (0, NCKMAX - NCK0), (0, 0))
    cpad1 = ((0, 0), (0, NCKMAX - NCK1), (0, 0))
    srcp = jnp.concatenate(
        [jnp.pad(srcf[:e0].reshape(NS, NCK0, CH), cpad0),
         jnp.pad(srcf[e0:].reshape(NS, NCK1, CH), cpad1)], axis=0)
    dstp = jnp.concatenate(
        [jnp.pad(dstf[:e0].reshape(NS, NCK0, CH), cpad0),
         jnp.pad(dstf[e0:].reshape(NS, NCK1, CH), cpad1)], axis=0)
    xp = jnp.pad(x, ((0, NPAD - NN), (0, 0)))

    degt = _get_deg_kernel()(dstp_deg)
    dis, u1 = _prep(degt, xp, W1)
    _seg = _get_seg_kernel()
    s1 = _seg(u1, srcp, dstp)
    u2 = _mid(s1, u1, dis, b1, g1, be1, a1, W2)
    s2 = _seg(u2, srcp, dstp)
    u3 = _mid(s2, u2, dis, b2, g2, be2, a2, W3)
    s3 = _seg(u3, srcp, dstp)
    return _final(s3, u3, dis, b3)


# speedup vs baseline: 1.3451x; 1.3451x over previous
import functools

import jax
import jax.numpy as jnp
from jax import lax
from jax.experimental import pallas as pl
from jax.experimental.pallas import tpu as pltpu
from jax.experimental.pallas import tpu_sc as plsc

NN = 10000
EE = 320000
DD = 128
NC = 2
NS = 16
NW = NC * NS
CH = 128
NCHUNK = 79
EPT = CH * NCHUNK
EPAD = EPT * NW
NPAD = 10240
ROWS_PER_TILE = NPAD // NS
DUMMY = NN
DW = 128


@functools.lru_cache(maxsize=None)
def _get_mesh():
    return plsc.VectorSubcoreMesh(core_axis_name="c", subcore_axis_name="s",
                                  num_cores=NC, num_subcores=NS)


def _deg_body(dst_hbm, out_hbm, dst_v, obuf, zbuf, acc):
    c = lax.axis_index("c")
    s = lax.axis_index("s")
    wid = c * NS + s

    def obody(r, carry):
        for q in range(DW // 16):
            obuf[r, pl.ds(q * 16, 16)] = jnp.ones((16,), jnp.float32)
        return carry

    lax.fori_loop(0, CH, obody, 0)
    for r in range(16):
        for q in range(DW // 16):
            zbuf[r, pl.ds(q * 16, 16)] = jnp.zeros((16,), jnp.float32)
    base = s * ROWS_PER_TILE

    def zbody(k, carry):
        pltpu.sync_copy(zbuf, acc.at[pl.ds(base + k * 16, 16)])
        return carry

    lax.fori_loop(0, ROWS_PER_TILE // 16, zbody, 0)
    pltpu.sync_copy(dst_hbm.at[wid], dst_v)
    plsc.subcore_barrier()

    def body(j, carry):
        pltpu.sync_copy(obuf, acc.at[dst_v.at[j]], add=True)
        return carry

    lax.fori_loop(0, NCHUNK, body, 0)
    plsc.subcore_barrier()
    pltpu.sync_copy(acc.at[pl.ds(base, ROWS_PER_TILE)],
                    out_hbm.at[c, pl.ds(base, ROWS_PER_TILE)])


@functools.lru_cache(maxsize=None)
def _get_deg_kernel():
    return pl.kernel(
        _deg_body,
        out_type=jax.ShapeDtypeStruct((NC, NPAD, DW), jnp.float32),
        mesh=_get_mesh(),
        scratch_types=[
            pltpu.VMEM((NCHUNK, CH), jnp.int32),
            pltpu.VMEM((CH, DW), jnp.float32),
            pltpu.VMEM((16, DW), jnp.float32),
            pltpu.VMEM_SHARED((NPAD, DW), jnp.float32),
        ],
    )


def _seg_body(u_hbm, src_hbm, dst_hbm, out_hbm, src_v, dst_v, gbuf, zbuf,
              acc, sem):
    c = lax.axis_index("c")
    s = lax.axis_index("s")
    wid = c * NS + s
    for r in range(16):
        for q in range(DD // 16):
            zbuf[r, pl.ds(q * 16, 16)] = jnp.zeros((16,), jnp.float32)
    base = s * ROWS_PER_TILE

    def zbody(k, carry):
        pltpu.sync_copy(zbuf, acc.at[pl.ds(base + k * 16, 16)])
        return carry

    lax.fori_loop(0, ROWS_PER_TILE // 16, zbody, 0)
    pltpu.sync_copy(src_hbm.at[wid], src_v)
    pltpu.sync_copy(dst_hbm.at[wid], dst_v)
    plsc.subcore_barrier()

    def body(j, carry):
        pltpu.async_copy(u_hbm.at[src_v.at[j]], gbuf, sem).wait()
        pltpu.sync_copy(gbuf, acc.at[dst_v.at[j]], add=True)
        return carry

    lax.fori_loop(0, NCHUNK, body, 0)
    plsc.subcore_barrier()
    pltpu.sync_copy(acc.at[pl.ds(base, ROWS_PER_TILE)],
                    out_hbm.at[c, pl.ds(base, ROWS_PER_TILE)])


@functools.lru_cache(maxsize=None)
def _get_seg_kernel():
    return pl.kernel(
        _seg_body,
        out_type=jax.ShapeDtypeStruct((NC, NPAD, DD), jnp.float32),
        mesh=_get_mesh(),
        scratch_types=[
            pltpu.VMEM((NCHUNK, CH), jnp.int32),
            pltpu.VMEM((NCHUNK, CH), jnp.int32),
            pltpu.VMEM((CH, DD), jnp.float32),
            pltpu.VMEM((16, DD), jnp.float32),
            pltpu.VMEM_SHARED((NPAD, DD), jnp.float32),
            pltpu.SemaphoreType.DMA,
        ],
    )


def _prep_body(deg_ref, x_ref, w_ref, dis_ref, u_ref):
    deg = deg_ref[0, :, 0:1] + deg_ref[1, :, 0:1] + 1.0
    row = lax.broadcasted_iota(jnp.int32, (NPAD, 1), 0)
    dis = jnp.where(row < NN, lax.rsqrt(deg), 0.0)
    dis_ref[...] = dis
    u_ref[...] = dis * jnp.dot(x_ref[...], w_ref[...],
                               preferred_element_type=jnp.float32)


_prep = pl.pallas_call(
    _prep_body,
    out_shape=(jax.ShapeDtypeStruct((NPAD, 1), jnp.float32),
               jax.ShapeDtypeStruct((NPAD, DD), jnp.float32)),
)


def _mid_body(s_ref, u_ref, dis_ref, b_ref, g_ref, be_ref, a_ref, w_ref,
              out_ref):
    dis = dis_ref[...]
    y = dis * (s_ref[0] + s_ref[1] + u_ref[...]) + b_ref[...]
    row = lax.broadcasted_iota(jnp.int32, (NPAD, 1), 0)
    mask = row < NN
    ym = jnp.where(mask, y, 0.0)
    m = jnp.sum(ym, axis=0, keepdims=True) * (1.0 / NN)
    d = jnp.where(mask, y - m, 0.0)
    v = jnp.sum(d * d, axis=0, keepdims=True) * (1.0 / NN)
    z = g_ref[...] * d * lax.rsqrt(v + 1e-5) + be_ref[...]
    a = a_ref[0]
    z = jnp.maximum(z, 0.0) + a * jnp.minimum(z, 0.0)
    out_ref[...] = dis * jnp.dot(z, w_ref[...],
                                 preferred_element_type=jnp.float32)


_mid = pl.pallas_call(
    _mid_body,
    out_shape=jax.ShapeDtypeStruct((NPAD, DD), jnp.float32),
)


def _final_body(s_ref, u_ref, dis_ref, b_ref, out_ref):
    y = dis_ref[...] * (s_ref[0] + s_ref[1] + u_ref[...]) + b_ref[...]
    out_ref[...] = y[:NN, :]


_final = pl.pallas_call(
    _final_body,
    out_shape=jax.ShapeDtypeStruct((NN, DD), jnp.float32),
)


def kernel(x, edge_index, W1, b1, g1, be1, a1, W2, b2, g2, be2, a2, W3, b3):
    src = edge_index[0]
    dst = edge_index[1]
    pad = EPAD - EE
    srcp = jnp.concatenate(
        [src, jnp.zeros((pad,), jnp.int32)]).reshape(NW, NCHUNK, CH)
    dstp = jnp.concatenate(
        [dst, DUMMY + (jnp.arange(pad, dtype=jnp.int32) % (NPAD - NN))]
    ).reshape(NW, NCHUNK, CH)
    xp = jnp.pad(x, ((0, NPAD - NN), (0, 0)))

    degt = _get_deg_kernel()(dstp)
    dis, u1 = _prep(degt, xp, W1)
    _seg = _get_seg_kernel()
    s1 = _seg(u1, srcp, dstp)
    u2 = _mid(s1, u1, dis, b1, g1, be1, a1, W2)
    s2 = _seg(u2, srcp, dstp)
    u3 = _mid(s2, u2, dis, b2, g2, be2, a2, W3)
    s3 = _seg(u3, srcp, dstp)
    return _final(s3, u3, dis, b3)
